# Initial kernel scaffold; baseline (speedup 1.0000x reference)
import jax
import jax.numpy as jnp
from jax.experimental import pallas as pl


def kernel(input, flow, attention):
    raise NotImplementedError("write your pallas kernel here")


# speedup vs baseline: 3.7200x; 3.7200x over previous
import jax
import jax.numpy as jnp
from jax import lax
from jax.experimental import pallas as pl
from jax.experimental.pallas import tpu as pltpu
from jax.experimental.pallas import tpu_sc as plsc

_B, _C, _H, _W, _K = 2, 96, 224, 224, 2
_CP = 128
_HW = _H * _W
_N = _B * _HW
_NC, _NS, _L = 2, 16, 16
_NW = _NC * _NS
_ROWS_PER = _N // _NW
_CHUNK = 112
_NCHUNK = _ROWS_PER // _CHUNK


def _body(inp_hbm, flow_hbm, att_hbm, out_hbm,
          fx, fy, av, ia, ib, ic, id_, wa, wb, wc, wd,
          ra, rb, rc, rd, ov, sem):
    wid = lax.axis_index("s") * _NC + lax.axis_index("c")
    b = wid // 16
    wloc = wid - b * 16
    imgbase = b * _HW

    def chunk_body(ci, carry):
        off = wloc * _ROWS_PER + ci * _CHUNK
        rows0 = imgbase + off
        yrow = wloc * 14 + ci // 2
        xc0 = (ci % 2) * _CHUNK
        yrow_f = yrow.astype(jnp.float32)

        for k in range(_K):
            fbase = b * (2 * _K * _HW) + off
            abase = b * (_K * _HW) + off
            pltpu.sync_copy(flow_hbm.at[pl.ds(fbase + 2 * k * _HW, _CHUNK)], fx)
            pltpu.sync_copy(flow_hbm.at[pl.ds(fbase + (2 * k + 1) * _HW, _CHUNK)], fy)
            pltpu.sync_copy(att_hbm.at[pl.ds(abase + k * _HW, _CHUNK)], av)

            for j in range(_CHUNK // _L):
                sl = pl.ds(j * _L, _L)
                xc = xc0 + j * _L + lax.iota(jnp.int32, _L)
                x = jnp.clip(xc.astype(jnp.float32) + fx[sl], 0.0, _W - 1.0)
                y = jnp.clip(yrow_f + fy[sl], 0.0, _H - 1.0)
                x0 = x.astype(jnp.int32)
                y0 = y.astype(jnp.int32)
                dx = x - x0.astype(jnp.float32)
                dy = y - y0.astype(jnp.float32)
                x1 = jnp.minimum(x0 + 1, _W - 1)
                y1 = jnp.minimum(y0 + 1, _H - 1)
                ry0 = imgbase + y0 * _W
                ry1 = imgbase + y1 * _W
                ia[sl] = ry0 + x0
                ib[sl] = ry1 + x0
                ic[sl] = ry0 + x1
                id_[sl] = ry1 + x1
                a_v = av[sl]
                omdx = 1.0 - dx
                omdy = 1.0 - dy
                wa[sl] = omdx * omdy * a_v
                wb[sl] = omdx * dy * a_v
                wc[sl] = dx * omdy * a_v
                wd[sl] = dx * dy * a_v

            cps = [pltpu.async_copy(inp_hbm.at[ia], ra, sem),
                   pltpu.async_copy(inp_hbm.at[ib], rb, sem),
                   pltpu.async_copy(inp_hbm.at[ic], rc, sem),
                   pltpu.async_copy(inp_hbm.at[id_], rd, sem)]
            for cp in cps:
                cp.wait()

            def fma_body(p, _):
                pv = jnp.full((_L,), p, dtype=jnp.int32)
                was = plsc.load_gather(wa, [pv])
                wbs = plsc.load_gather(wb, [pv])
                wcs = plsc.load_gather(wc, [pv])
                wds = plsc.load_gather(wd, [pv])
                for cc in range(_C // _L):
                    cs = pl.ds(cc * _L, _L)
                    contrib = (was * ra[p, cs] + wbs * rb[p, cs] +
                               wcs * rc[p, cs] + wds * rd[p, cs])
                    if k == 0:
                        ov[p, cs] = contrib
                    else:
                        ov[p, cs] = ov[p, cs] + contrib
                return _

            lax.fori_loop(0, _CHUNK, fma_body, None)

        pltpu.sync_copy(ov, out_hbm.at[pl.ds(rows0, _CHUNK), :])
        return carry

    lax.fori_loop(0, _NCHUNK, chunk_body, None)


def _warp_sc(inp_t, flow_r, att_r):
    mesh = plsc.VectorSubcoreMesh(core_axis_name="c", subcore_axis_name="s")
    return pl.kernel(
        _body,
        out_type=jax.ShapeDtypeStruct((_N, _C), jnp.float32),
        mesh=mesh,
        compiler_params=pltpu.CompilerParams(needs_layout_passes=False),
        scratch_types=[
            pltpu.VMEM((_CHUNK,), jnp.float32),
            pltpu.VMEM((_CHUNK,), jnp.float32),
            pltpu.VMEM((_CHUNK,), jnp.float32),
            pltpu.VMEM((_CHUNK,), jnp.int32),
            pltpu.VMEM((_CHUNK,), jnp.int32),
            pltpu.VMEM((_CHUNK,), jnp.int32),
            pltpu.VMEM((_CHUNK,), jnp.int32),
            pltpu.VMEM((_CHUNK,), jnp.float32),
            pltpu.VMEM((_CHUNK,), jnp.float32),
            pltpu.VMEM((_CHUNK,), jnp.float32),
            pltpu.VMEM((_CHUNK,), jnp.float32),
            pltpu.VMEM((_CHUNK, _CP), jnp.float32),
            pltpu.VMEM((_CHUNK, _CP), jnp.float32),
            pltpu.VMEM((_CHUNK, _CP), jnp.float32),
            pltpu.VMEM((_CHUNK, _CP), jnp.float32),
            pltpu.VMEM((_CHUNK, _C), jnp.float32),
            pltpu.SemaphoreType.DMA,
        ],
    )(inp_t, flow_r, att_r)


def kernel(input, flow, attention):
    inp_t = jnp.transpose(input, (0, 2, 3, 1)).reshape(_N, _C)
    inp_t = jnp.pad(inp_t, ((0, 0), (0, _CP - _C)))
    flow_r = flow.reshape(_B * 2 * _K * _HW)
    att_r = attention.reshape(_B * _K * _HW)
    out_t = _warp_sc(inp_t, flow_r, att_r)
    return jnp.transpose(out_t.reshape(_B, _H, _W, _C), (0, 3, 1, 2))
